# initial kernel scaffold (unmeasured)
import jax
import jax.numpy as jnp
from jax import lax
from jax.experimental import pallas as pl
from jax.experimental.pallas import tpu as pltpu


def kernel(
    x,
):
    def body(*refs):
        pass

    out_shape = jax.ShapeDtypeStruct(..., jnp.float32)
    return pl.pallas_call(body, out_shape=out_shape)(...)


# baseline (device time: 14562 ns/iter reference)
import jax
import jax.numpy as jnp
from jax import lax
from jax.experimental import pallas as pl
from jax.experimental.pallas import tpu as pltpu

N_DEV = 8


def kernel(x):
    _, m, n_tot = x.shape
    c = n_tot // N_DEV

    def body(x_ref, out_ref, stage_ref, recv_ref, send_sems, recv_sems):
        my = lax.axis_index("i")

        for j in range(N_DEV):
            stage_ref[j] = x_ref[0, :, j * c:(j + 1) * c].astype(jnp.bfloat16)

        barrier = pltpu.get_barrier_semaphore()
        for k in range(1, N_DEV):
            peer = lax.rem(my + k, N_DEV)
            pl.semaphore_signal(
                barrier, inc=1,
                device_id=(peer,), device_id_type=pl.DeviceIdType.MESH,
            )
        pl.semaphore_wait(barrier, N_DEV - 1)

        rdmas = []
        for k in range(1, N_DEV):
            dest = lax.rem(my + k, N_DEV)
            rdma = pltpu.make_async_remote_copy(
                src_ref=stage_ref.at[dest],
                dst_ref=recv_ref.at[my],
                send_sem=send_sems.at[k - 1],
                recv_sem=recv_sems.at[my],
                device_id=(dest,),
                device_id_type=pl.DeviceIdType.MESH,
            )
            rdma.start()
            rdmas.append(rdma)

        acc = stage_ref[my].astype(jnp.float32)
        for k in range(1, N_DEV):
            src = lax.rem(my + k, N_DEV)
            recv_wait = pltpu.make_async_remote_copy(
                src_ref=stage_ref.at[0],
                dst_ref=recv_ref.at[src],
                send_sem=send_sems.at[0],
                recv_sem=recv_sems.at[src],
                device_id=(src,),
                device_id_type=pl.DeviceIdType.MESH,
            )
            recv_wait.wait_recv()
            acc = acc + recv_ref[src].astype(jnp.float32)
        out_ref[:, :] = acc

        for rdma in rdmas:
            rdma.wait_send()

    return pl.pallas_call(
        body,
        out_shape=jax.ShapeDtypeStruct((m, c), jnp.float32),
        in_specs=[pl.BlockSpec(memory_space=pltpu.VMEM)],
        out_specs=pl.BlockSpec(memory_space=pltpu.VMEM),
        scratch_shapes=[
            pltpu.VMEM((N_DEV, m, c), jnp.bfloat16),
            pltpu.VMEM((N_DEV, m, c), jnp.bfloat16),
            pltpu.SemaphoreType.DMA((N_DEV - 1,)),
            pltpu.SemaphoreType.DMA((N_DEV,)),
        ],
        compiler_params=pltpu.CompilerParams(collective_id=0),
    )(x)


# device time: 14220 ns/iter; 1.0241x vs baseline; 1.0241x over previous
import jax
import jax.numpy as jnp
from jax import lax
from jax.experimental import pallas as pl
from jax.experimental.pallas import tpu as pltpu

N_DEV = 8


def kernel(x):
    _, m, n_tot = x.shape
    c = n_tot // N_DEV

    def body(x_ref, out_ref, stage_ref, recv_ref, send_sems, recv_sems):
        my = lax.axis_index("i")

        barrier = pltpu.get_barrier_semaphore()
        for k in range(1, N_DEV):
            peer = lax.rem(my + k, N_DEV)
            pl.semaphore_signal(
                barrier, inc=1,
                device_id=(peer,), device_id_type=pl.DeviceIdType.MESH,
            )

        stage_ref[0] = x_ref[0, :, 0:c].astype(jnp.bfloat16)
        pl.semaphore_wait(barrier, N_DEV - 1)

        def _send(j):
            rdma = pltpu.make_async_remote_copy(
                src_ref=stage_ref.at[j],
                dst_ref=recv_ref.at[my],
                send_sem=send_sems.at[j],
                recv_sem=recv_sems.at[my],
                device_id=(j,),
                device_id_type=pl.DeviceIdType.MESH,
            )
            rdma.start()

        for j in range(N_DEV):
            if j > 0:
                stage_ref[j] = x_ref[0, :, j * c:(j + 1) * c].astype(
                    jnp.bfloat16)
            pl.when(j != my)(lambda j=j: _send(j))

        acc = stage_ref[my].astype(jnp.float32)
        for k in range(1, N_DEV):
            src = lax.rem(my + k, N_DEV)
            recv_wait = pltpu.make_async_remote_copy(
                src_ref=stage_ref.at[0],
                dst_ref=recv_ref.at[src],
                send_sem=send_sems.at[0],
                recv_sem=recv_sems.at[src],
                device_id=(src,),
                device_id_type=pl.DeviceIdType.MESH,
            )
            recv_wait.wait_recv()
            acc = acc + recv_ref[src].astype(jnp.float32)
        out_ref[:, :] = acc

        def _wait_send(j):
            pltpu.make_async_remote_copy(
                src_ref=stage_ref.at[j],
                dst_ref=recv_ref.at[my],
                send_sem=send_sems.at[j],
                recv_sem=recv_sems.at[my],
                device_id=(j,),
                device_id_type=pl.DeviceIdType.MESH,
            ).wait_send()

        for j in range(N_DEV):
            pl.when(j != my)(lambda j=j: _wait_send(j))

    return pl.pallas_call(
        body,
        out_shape=jax.ShapeDtypeStruct((m, c), jnp.float32),
        in_specs=[pl.BlockSpec(memory_space=pltpu.VMEM)],
        out_specs=pl.BlockSpec(memory_space=pltpu.VMEM),
        scratch_shapes=[
            pltpu.VMEM((N_DEV, m, c), jnp.bfloat16),
            pltpu.VMEM((N_DEV, m, c), jnp.bfloat16),
            pltpu.SemaphoreType.DMA((N_DEV,)),
            pltpu.SemaphoreType.DMA((N_DEV,)),
        ],
        compiler_params=pltpu.CompilerParams(collective_id=0),
    )(x)
